# Initial kernel scaffold; baseline (speedup 1.0000x reference)
import jax
import jax.numpy as jnp
from jax.experimental import pallas as pl


def kernel(edge, features, W1, b1, Wh, bh, W2, b2):
    raise NotImplementedError("write your pallas kernel here")


# speedup vs baseline: 11.1707x; 11.1707x over previous
import functools

import jax
import jax.numpy as jnp
from jax import lax
from jax.experimental import pallas as pl
from jax.experimental.pallas import tpu as pltpu
from jax.experimental.pallas import tpu_sc as plsc

N = 10000
E = 320000
_NC, _NS = 2, 16
_TILES = _NC * _NS
_EPT = E // _TILES
_CHUNK = 80
_NCHUNK = _EPT // _CHUNK
_ROWS_LO = 624
_ROWS_HI = 640


def _make_agg(D, gather):
    mesh = plsc.VectorSubcoreMesh(core_axis_name="c", subcore_axis_name="s")

    @functools.partial(
        pl.kernel,
        mesh=mesh,
        out_type=jax.ShapeDtypeStruct((_NC, N, D), jnp.float32),
        scratch_types=[
            pltpu.VMEM_SHARED((N, D), jnp.float32),
            pltpu.VMEM((_CHUNK,), jnp.int32),
            pltpu.VMEM((_CHUNK,), jnp.int32),
            pltpu.VMEM((_CHUNK, D), jnp.float32),
            pltpu.SemaphoreType.DMA,
        ],
    )
    def agg(src_hbm, dst_hbm, y_hbm, out_hbm, acc, idx_s, idx_d, rows, sem):
        c = lax.axis_index("c")
        s = lax.axis_index("s")
        tile = c * _NS + s

        if gather:
            @pl.when(s < _NS - 1)
            def _():
                pltpu.sync_copy(y_hbm.at[pl.ds(s * _ROWS_LO, _ROWS_LO)],
                                acc.at[pl.ds(s * _ROWS_LO, _ROWS_LO)])

            @pl.when(s == _NS - 1)
            def _():
                pltpu.sync_copy(y_hbm.at[pl.ds(N - _ROWS_HI, _ROWS_HI)],
                                acc.at[pl.ds(N - _ROWS_HI, _ROWS_HI)])
        else:
            for j in range(_CHUNK):
                rows[j, :] = jnp.ones((16,), jnp.float32)
            for j in range(8):
                base = pl.multiple_of(s * _ROWS_LO + j * _CHUNK, 8)
                pltpu.sync_copy(rows, acc.at[pl.ds(base, _CHUNK)])

        plsc.subcore_barrier()

        base0 = tile * _EPT

        def body(i, carry):
            base = pl.multiple_of(base0 + i * _CHUNK, 8)
            pltpu.sync_copy(dst_hbm.at[pl.ds(base, _CHUNK)], idx_d)
            if gather:
                pltpu.sync_copy(src_hbm.at[pl.ds(base, _CHUNK)], idx_s)
                pltpu.async_copy(y_hbm.at[idx_s], rows, sem).wait()
            pltpu.sync_copy(rows, acc.at[idx_d], add=True)
            return carry

        lax.fori_loop(0, _NCHUNK, body, 0)

        plsc.subcore_barrier()

        @pl.when(s < _NS - 1)
        def _():
            pltpu.sync_copy(acc.at[pl.ds(s * _ROWS_LO, _ROWS_LO)],
                            out_hbm.at[c, pl.ds(s * _ROWS_LO, _ROWS_LO)])

        @pl.when(s == _NS - 1)
        def _():
            pltpu.sync_copy(acc.at[pl.ds(N - _ROWS_HI, _ROWS_HI)],
                            out_hbm.at[c, pl.ds(N - _ROWS_HI, _ROWS_HI)])

    return agg


_R = 1000


def _tc_first(degp, x, W):
    def body(degp_ref, x_ref, w_ref, y_ref, dinv_ref):
        deg = degp_ref[0, :, 0:1] + degp_ref[1, :, 0:1] - 1.0
        dinv = lax.rsqrt(deg)
        dinv_ref[...] = dinv
        y_ref[...] = dinv * jnp.dot(x_ref[...], w_ref[...],
                                    preferred_element_type=jnp.float32)

    return pl.pallas_call(
        body,
        grid=(N // _R,),
        in_specs=[
            pl.BlockSpec((2, _R, 16), lambda i: (0, i, 0)),
            pl.BlockSpec((_R, 128), lambda i: (i, 0)),
            pl.BlockSpec((128, 128), lambda i: (0, 0)),
        ],
        out_specs=[
            pl.BlockSpec((_R, 128), lambda i: (i, 0)),
            pl.BlockSpec((_R, 1), lambda i: (i, 0)),
        ],
        out_shape=[
            jax.ShapeDtypeStruct((N, 128), jnp.float32),
            jax.ShapeDtypeStruct((N, 1), jnp.float32),
        ],
    )(degp, x, W)


def _tc_mid(parts, y_prev, dinv, b, W, d_in, d_out):
    def body(p_ref, y_ref, dinv_ref, b_ref, w_ref, o_ref):
        dinv = dinv_ref[...]
        h = dinv * (p_ref[0] + p_ref[1] - y_ref[...]) + b_ref[...]
        h = jnp.maximum(h, 0.0)
        o_ref[...] = dinv * jnp.dot(h, w_ref[...],
                                    preferred_element_type=jnp.float32)

    return pl.pallas_call(
        body,
        grid=(N // _R,),
        in_specs=[
            pl.BlockSpec((2, _R, d_in), lambda i: (0, i, 0)),
            pl.BlockSpec((_R, d_in), lambda i: (i, 0)),
            pl.BlockSpec((_R, 1), lambda i: (i, 0)),
            pl.BlockSpec((1, d_in), lambda i: (0, 0)),
            pl.BlockSpec((d_in, d_out), lambda i: (0, 0)),
        ],
        out_specs=pl.BlockSpec((_R, d_out), lambda i: (i, 0)),
        out_shape=jax.ShapeDtypeStruct((N, d_out), jnp.float32),
    )(parts, y_prev, dinv, b, W)


def _tc_last(parts, y_prev, dinv, b, d_in, d_pad):
    def body(p_ref, y_ref, dinv_ref, b_ref, o_ref):
        z = dinv_ref[...] * (p_ref[0, :, 0:d_in] + p_ref[1, :, 0:d_in]
                             - y_ref[:, 0:d_in]) + b_ref[...]
        m = jnp.max(z, axis=1, keepdims=True)
        zm = z - m
        o_ref[...] = zm - jnp.log(jnp.sum(jnp.exp(zm), axis=1, keepdims=True))

    return pl.pallas_call(
        body,
        grid=(N // _R,),
        in_specs=[
            pl.BlockSpec((2, _R, d_pad), lambda i: (0, i, 0)),
            pl.BlockSpec((_R, d_pad), lambda i: (i, 0)),
            pl.BlockSpec((_R, 1), lambda i: (i, 0)),
            pl.BlockSpec((1, d_in), lambda i: (0, 0)),
        ],
        out_specs=pl.BlockSpec((_R, d_in), lambda i: (i, 0)),
        out_shape=jax.ShapeDtypeStruct((N, d_in), jnp.float32),
    )(parts, y_prev, dinv, b)


def kernel(edge, features, W1, b1, Wh, bh, W2, b2):
    edge = edge.astype(jnp.int32)
    src, dst = edge[0], edge[1]
    dummy16 = jnp.zeros((N, 16), jnp.float32)

    degp = _make_agg(16, gather=False)(src, dst, dummy16)
    y1, dinv = _tc_first(degp, features, W1)

    p1 = _make_agg(128, gather=True)(src, dst, y1)
    y2 = _tc_mid(p1, y1, dinv, b1.reshape(1, -1), Wh, 128, 128)

    W2p = jnp.concatenate([W2, jnp.zeros((128, 64), jnp.float32)], axis=1)
    p2 = _make_agg(128, gather=True)(src, dst, y2)
    y3 = _tc_mid(p2, y2, dinv, bh.reshape(1, -1), W2p, 128, 128)

    p3 = _make_agg(128, gather=True)(src, dst, y3)
    return _tc_last(p3, y3, dinv, b2.reshape(1, -1), 64, 128)
